# baseline (speedup 1.0000x reference)
import jax
import jax.numpy as jnp
from jax import lax
from jax.experimental import pallas as pl
from jax.experimental.pallas import tpu as pltpu
from jax.experimental.pallas import tpu_sc as plsc

B, N, S = 8, 4096, 1024
D1, D2 = 128, 256
H1, H2 = 256, 128
NT = 512

NW = 32
PPW = B * N // NW
G = 32
NCH = PPW // G


def _top3_kernel(x1_ref, x2_ref, i0_ref, i1_ref, i2_ref,
                 w0_ref, w1_ref, w2_ref):
    b = pl.program_id(0)
    x1 = x1_ref[0]
    x2 = x2_ref[0]
    sqn1 = jnp.sum(x1 * x1, axis=1, keepdims=True)
    sqn2 = jnp.sum(x2 * x2, axis=0, keepdims=True)
    dot = jnp.dot(x1, x2, preferred_element_type=jnp.float32)
    dist = sqn1 + sqn2 - 2.0 * dot

    lane = jax.lax.broadcasted_iota(jnp.int32, (NT, S), 1)
    big = jnp.float32(jnp.inf)

    def take_min(d):
        m = jnp.min(d, axis=1, keepdims=True)
        i = jnp.min(jnp.where(d == m, lane, S), axis=1, keepdims=True)
        d = jnp.where(lane == i, big, d)
        return m, i, d

    m1, i1, dist = take_min(dist)
    m2, i2, dist = take_min(dist)
    m3, i3, dist = take_min(dist)

    r1 = 1.0 / (m1 + 1e-8)
    r2 = 1.0 / (m2 + 1e-8)
    r3 = 1.0 / (m3 + 1e-8)
    norm = r1 + r2 + r3

    base = b * S
    i0_ref[0] = i1 + base
    i1_ref[0] = i2 + base
    i2_ref[0] = i3 + base
    w0_ref[0] = jnp.broadcast_to(r1 / norm, (NT, 16))
    w1_ref[0] = jnp.broadcast_to(r2 / norm, (NT, 16))
    w2_ref[0] = jnp.broadcast_to(r3 / norm, (NT, 16))


def _interp_sc(i0h, i1h, i2h, w0h, w1h, w2h, table, outh,
               i0v, i1v, i2v, w0v, w1v, w2v,
               r0a, r1a, r2a, r0b, r1b, r2b, ova, ovb,
               gsa, gsb, osa, osb):
    wid = lax.axis_index("s") * 2 + lax.axis_index("c")
    wbase = wid * PPW

    pltpu.sync_copy(i0h.at[pl.ds(wbase, PPW)], i0v)
    pltpu.sync_copy(i1h.at[pl.ds(wbase, PPW)], i1v)
    pltpu.sync_copy(i2h.at[pl.ds(wbase, PPW)], i2v)
    pltpu.sync_copy(w0h.at[pl.ds(wbase * 16, PPW * 16)], w0v)
    pltpu.sync_copy(w1h.at[pl.ds(wbase * 16, PPW * 16)], w1v)
    pltpu.sync_copy(w2h.at[pl.ds(wbase * 16, PPW * 16)], w2v)

    def fire(it, r0, r1, r2, sem):
        off = it * G
        pltpu.async_copy(table.at[i0v.at[pl.ds(off, G)]], r0, sem)
        pltpu.async_copy(table.at[i1v.at[pl.ds(off, G)]], r1, sem)
        pltpu.async_copy(table.at[i2v.at[pl.ds(off, G)]], r2, sem)

    def drain(r0, r1, r2, sem):
        pltpu.make_async_copy(table.at[i0v.at[pl.ds(0, G)]], r0, sem).wait()
        pltpu.make_async_copy(table.at[i1v.at[pl.ds(0, G)]], r1, sem).wait()
        pltpu.make_async_copy(table.at[i2v.at[pl.ds(0, G)]], r2, sem).wait()

    def compute(it, r0, r1, r2, ov):
        off = it * G
        for g in range(G):
            a0 = w0v[pl.ds((off + g) * 16, 16)]
            a1 = w1v[pl.ds((off + g) * 16, 16)]
            a2 = w2v[pl.ds((off + g) * 16, 16)]
            for c in range(D2 // 16):
                sl = pl.ds(c * 16, 16)
                ov[g, sl] = (a0 * r0[g, sl] + a1 * r1[g, sl]
                             + a2 * r2[g, sl])

    fire(0, r0a, r1a, r2a, gsa)
    fire(1, r0b, r1b, r2b, gsb)

    def half(j, it, r0, r1, r2, ov, gsem, osem):
        drain(r0, r1, r2, gsem)

        @pl.when(j > 0)
        def _():
            pltpu.make_async_copy(ov, outh.at[pl.ds(0, G)], osem).wait()
        compute(it, r0, r1, r2, ov)
        pltpu.async_copy(ov, outh.at[pl.ds(wbase + it * G, G)], osem)

        @pl.when(it + 2 < NCH)
        def _():
            fire(it + 2, r0, r1, r2, gsem)

    def body(j, carry):
        half(j, 2 * j, r0a, r1a, r2a, ova, gsa, osa)
        half(j, 2 * j + 1, r0b, r1b, r2b, ovb, gsb, osb)
        return carry

    lax.fori_loop(0, NCH // 2, body, 0)
    pltpu.make_async_copy(ova, outh.at[pl.ds(0, G)], osa).wait()
    pltpu.make_async_copy(ovb, outh.at[pl.ds(0, G)], osb).wait()


def _mlp_kernel(p1_ref, it_ref, w1_ref, b1_ref, w2_ref, b2_ref, out_ref):
    p1 = p1_ref[0]
    itp = it_ref[0]
    w1a = w1_ref[:, :D1]
    w1b = w1_ref[:, D1:]
    hb = jax.lax.dot_general(w1b, itp, (((1,), (1,)), ((), ())),
                             preferred_element_type=jnp.float32)
    h = (jnp.dot(w1a, p1, preferred_element_type=jnp.float32)
         + hb + b1_ref[:, :1])
    h = jnp.maximum(h, 0.0)
    out = jnp.dot(w2_ref[...], h, preferred_element_type=jnp.float32) + b2_ref[:, :1]
    out_ref[0] = jnp.maximum(out, 0.0)


def kernel(xyz1, xyz2, points1, points2, W1, b1, W2, b2):
    x1t = jnp.transpose(xyz1, (0, 2, 1))
    x1t = jnp.pad(x1t, ((0, 0), (0, 0), (0, 5)))
    x2p = jnp.pad(xyz2, ((0, 0), (0, 5), (0, 0)))

    grid = (B, N // NT)
    idx_spec = pl.BlockSpec((1, NT, 1), lambda b, n: (b, n, 0))
    w_spec = pl.BlockSpec((1, NT, 16), lambda b, n: (b, n, 0))
    idx_sds = jax.ShapeDtypeStruct((B, N, 1), jnp.int32)
    w_sds = jax.ShapeDtypeStruct((B, N, 16), jnp.float32)
    i0, i1, i2, w0, w1, w2 = pl.pallas_call(
        _top3_kernel,
        grid=grid,
        in_specs=[
            pl.BlockSpec((1, NT, 8), lambda b, n: (b, n, 0)),
            pl.BlockSpec((1, 8, S), lambda b, n: (b, 0, 0)),
        ],
        out_specs=[idx_spec, idx_spec, idx_spec, w_spec, w_spec, w_spec],
        out_shape=[idx_sds, idx_sds, idx_sds, w_sds, w_sds, w_sds],
    )(x1t, x2p)

    i0 = jnp.reshape(i0, (B * N,))
    i1 = jnp.reshape(i1, (B * N,))
    i2 = jnp.reshape(i2, (B * N,))
    w0 = jnp.reshape(w0, (B * N * 16,))
    w1 = jnp.reshape(w1, (B * N * 16,))
    w2 = jnp.reshape(w2, (B * N * 16,))
    table = jnp.reshape(jnp.transpose(points2, (0, 2, 1)), (B * S, D2))

    mesh = plsc.VectorSubcoreMesh(core_axis_name="c", subcore_axis_name="s")
    interp = pl.kernel(
        _interp_sc,
        mesh=mesh,
        out_type=jax.ShapeDtypeStruct((B * N, D2), jnp.float32),
        scratch_types=[
            pltpu.VMEM((PPW,), jnp.int32),
            pltpu.VMEM((PPW,), jnp.int32),
            pltpu.VMEM((PPW,), jnp.int32),
            pltpu.VMEM((PPW * 16,), jnp.float32),
            pltpu.VMEM((PPW * 16,), jnp.float32),
            pltpu.VMEM((PPW * 16,), jnp.float32),
            pltpu.VMEM((G, D2), jnp.float32),
            pltpu.VMEM((G, D2), jnp.float32),
            pltpu.VMEM((G, D2), jnp.float32),
            pltpu.VMEM((G, D2), jnp.float32),
            pltpu.VMEM((G, D2), jnp.float32),
            pltpu.VMEM((G, D2), jnp.float32),
            pltpu.VMEM((G, D2), jnp.float32),
            pltpu.VMEM((G, D2), jnp.float32),
            pltpu.SemaphoreType.DMA,
            pltpu.SemaphoreType.DMA,
            pltpu.SemaphoreType.DMA,
            pltpu.SemaphoreType.DMA,
        ],
    )(i0, i1, i2, w0, w1, w2, table)
    interp = jnp.reshape(interp, (B, N, D2))

    b1c = jnp.reshape(b1, (H1, 1))
    b2c = jnp.reshape(b2, (H2, 1))
    out = pl.pallas_call(
        _mlp_kernel,
        grid=grid,
        in_specs=[
            pl.BlockSpec((1, D1, NT), lambda b, n: (b, 0, n)),
            pl.BlockSpec((1, NT, D2), lambda b, n: (b, n, 0)),
            pl.BlockSpec((H1, D1 + D2), lambda b, n: (0, 0)),
            pl.BlockSpec((H1, 1), lambda b, n: (0, 0)),
            pl.BlockSpec((H2, H1), lambda b, n: (0, 0)),
            pl.BlockSpec((H2, 1), lambda b, n: (0, 0)),
        ],
        out_specs=pl.BlockSpec((1, H2, NT), lambda b, n: (b, 0, n)),
        out_shape=jax.ShapeDtypeStruct((B, H2, N), jnp.float32),
    )(points1, interp, W1, b1c, W2, b2c)
    return out


# speedup vs baseline: 2.9001x; 2.9001x over previous
import jax
import jax.numpy as jnp
from jax.experimental import pallas as pl

B, N, S = 8, 4096, 1024
D1, D2 = 128, 256
H1, H2 = 256, 128
NT = 1024


def _fp_kernel(x1_ref, x2_ref, p1_ref, p2_ref, w1_ref, b1_ref, w2_ref, b2_ref,
               out_ref):
    x1 = x1_ref[0]
    x2 = x2_ref[0]
    sqn1 = jnp.sum(x1 * x1, axis=0, keepdims=True)
    sqn2 = jnp.sum(x2 * x2, axis=1, keepdims=True)
    dot = jnp.dot(x2, x1, preferred_element_type=jnp.float32)
    dist = sqn2 + sqn1 - 2.0 * dot

    big = jnp.float32(jnp.inf)

    def take_min(d):
        m = jnp.min(d, axis=0, keepdims=True)
        msk = d == m
        d = jnp.where(msk, big, d)
        return m, msk, d

    m1, k1, dist = take_min(dist)
    m2, k2, dist = take_min(dist)
    m3, k3, dist = take_min(dist)

    r1 = 1.0 / (m1 + 1e-8)
    r2 = 1.0 / (m2 + 1e-8)
    r3 = 1.0 / (m3 + 1e-8)
    norm = r1 + r2 + r3
    w1 = r1 / norm
    w2 = r2 / norm
    w3 = r3 / norm

    zero = jnp.zeros((S, NT), jnp.float32)
    O = (jnp.where(k1, w1, zero) + jnp.where(k2, w2, zero)
         + jnp.where(k3, w3, zero))

    p2 = p2_ref[0]
    interp = jnp.dot(p2, O, preferred_element_type=jnp.float32)

    p1 = p1_ref[0]
    w1a = w1_ref[:, :D1]
    w1b = w1_ref[:, D1:]
    h = (jnp.dot(w1a, p1, preferred_element_type=jnp.float32)
         + jnp.dot(w1b, interp, preferred_element_type=jnp.float32)
         + b1_ref[:, :1])
    h = jnp.maximum(h, 0.0)
    out = jnp.dot(w2_ref[...], h, preferred_element_type=jnp.float32) + b2_ref[:, :1]
    out_ref[0] = jnp.maximum(out, 0.0)


def kernel(xyz1, xyz2, points1, points2, W1, b1, W2, b2):
    x1p = jnp.pad(xyz1, ((0, 0), (0, 5), (0, 0)))
    x2t = jnp.transpose(xyz2, (0, 2, 1))
    x2t = jnp.pad(x2t, ((0, 0), (0, 0), (0, 5)))
    b1c = jnp.reshape(b1, (H1, 1))
    b2c = jnp.reshape(b2, (H2, 1))

    grid = (B, N // NT)
    out = pl.pallas_call(
        _fp_kernel,
        grid=grid,
        in_specs=[
            pl.BlockSpec((1, 8, NT), lambda b, n: (b, 0, n)),
            pl.BlockSpec((1, S, 8), lambda b, n: (b, 0, 0)),
            pl.BlockSpec((1, D1, NT), lambda b, n: (b, 0, n)),
            pl.BlockSpec((1, D2, S), lambda b, n: (b, 0, 0)),
            pl.BlockSpec((H1, D1 + D2), lambda b, n: (0, 0)),
            pl.BlockSpec((H1, 1), lambda b, n: (0, 0)),
            pl.BlockSpec((H2, H1), lambda b, n: (0, 0)),
            pl.BlockSpec((H2, 1), lambda b, n: (0, 0)),
        ],
        out_specs=pl.BlockSpec((1, H2, NT), lambda b, n: (b, 0, n)),
        out_shape=jax.ShapeDtypeStruct((B, H2, N), jnp.float32),
    )(x1p, x2t, points1, points2, W1, b1c, W2, b2c)
    return out


# speedup vs baseline: 3.2137x; 1.1081x over previous
import jax
import jax.numpy as jnp
from jax.experimental import pallas as pl

B, N, S = 8, 4096, 1024
D1, D2 = 128, 256
H1, H2 = 256, 128
NT = 2048


def _fp_kernel(x1_ref, x2_ref, p1_ref, p2_ref, w1_ref, b1_ref, w2_ref, b2_ref,
               out_ref):
    x1 = x1_ref[0]
    x2 = x2_ref[0]
    sqn1 = jnp.sum(x1 * x1, axis=0, keepdims=True)
    sqn2 = jnp.sum(x2 * x2, axis=1, keepdims=True)
    dot = jnp.dot(x2, x1, preferred_element_type=jnp.float32)
    dist = sqn2 + sqn1 - 2.0 * dot

    big = jnp.float32(jnp.inf)

    def take_min(d):
        m = jnp.min(d, axis=0, keepdims=True)
        msk = d == m
        d = jnp.where(msk, big, d)
        return m, msk, d

    m1, k1, dist = take_min(dist)
    m2, k2, dist = take_min(dist)
    m3, k3, dist = take_min(dist)

    r1 = 1.0 / (m1 + 1e-8)
    r2 = 1.0 / (m2 + 1e-8)
    r3 = 1.0 / (m3 + 1e-8)
    norm = r1 + r2 + r3
    w1 = r1 / norm
    w2 = r2 / norm
    w3 = r3 / norm

    zero = jnp.zeros((S, NT), jnp.float32)
    O = (jnp.where(k1, w1, zero) + jnp.where(k2, w2, zero)
         + jnp.where(k3, w3, zero))

    p2 = p2_ref[0]
    interp = jnp.dot(p2, O, preferred_element_type=jnp.float32)

    p1 = p1_ref[0]
    w1a = w1_ref[:, :D1]
    w1b = w1_ref[:, D1:]
    h = (jnp.dot(w1a, p1, preferred_element_type=jnp.float32)
         + jnp.dot(w1b, interp, preferred_element_type=jnp.float32)
         + b1_ref[:, :1])
    h = jnp.maximum(h, 0.0)
    out = jnp.dot(w2_ref[...], h, preferred_element_type=jnp.float32) + b2_ref[:, :1]
    out_ref[0] = jnp.maximum(out, 0.0)


def kernel(xyz1, xyz2, points1, points2, W1, b1, W2, b2):
    x1p = jnp.pad(xyz1, ((0, 0), (0, 5), (0, 0)))
    x2t = jnp.transpose(xyz2, (0, 2, 1))
    x2t = jnp.pad(x2t, ((0, 0), (0, 0), (0, 5)))
    b1c = jnp.reshape(b1, (H1, 1))
    b2c = jnp.reshape(b2, (H2, 1))

    grid = (B, N // NT)
    out = pl.pallas_call(
        _fp_kernel,
        grid=grid,
        in_specs=[
            pl.BlockSpec((1, 8, NT), lambda b, n: (b, 0, n)),
            pl.BlockSpec((1, S, 8), lambda b, n: (b, 0, 0)),
            pl.BlockSpec((1, D1, NT), lambda b, n: (b, 0, n)),
            pl.BlockSpec((1, D2, S), lambda b, n: (b, 0, 0)),
            pl.BlockSpec((H1, D1 + D2), lambda b, n: (0, 0)),
            pl.BlockSpec((H1, 1), lambda b, n: (0, 0)),
            pl.BlockSpec((H2, H1), lambda b, n: (0, 0)),
            pl.BlockSpec((H2, 1), lambda b, n: (0, 0)),
        ],
        out_specs=pl.BlockSpec((1, H2, NT), lambda b, n: (b, 0, n)),
        out_shape=jax.ShapeDtypeStruct((B, H2, N), jnp.float32),
    )(x1p, x2t, points1, points2, W1, b1c, W2, b2c)
    return out


# speedup vs baseline: 3.2553x; 1.0129x over previous
import jax
import jax.numpy as jnp
from jax.experimental import pallas as pl

B, N, S = 8, 4096, 1024
D1, D2 = 128, 256
H1, H2 = 256, 128
NT = 4096


def _fp_kernel(x1_ref, x2_ref, p1_ref, p2_ref, w1_ref, b1_ref, w2_ref, b2_ref,
               out_ref):
    x1 = x1_ref[0]
    x2 = x2_ref[0]
    sqn1 = jnp.sum(x1 * x1, axis=0, keepdims=True)
    sqn2 = jnp.sum(x2 * x2, axis=1, keepdims=True)
    dot = jnp.dot(x2, x1, preferred_element_type=jnp.float32)
    dist = sqn2 + sqn1 - 2.0 * dot

    big = jnp.float32(jnp.inf)

    def take_min(d):
        m = jnp.min(d, axis=0, keepdims=True)
        msk = d == m
        d = jnp.where(msk, big, d)
        return m, msk, d

    m1, k1, dist = take_min(dist)
    m2, k2, dist = take_min(dist)
    m3, k3, dist = take_min(dist)

    r1 = 1.0 / (m1 + 1e-8)
    r2 = 1.0 / (m2 + 1e-8)
    r3 = 1.0 / (m3 + 1e-8)
    norm = r1 + r2 + r3
    w1 = r1 / norm
    w2 = r2 / norm
    w3 = r3 / norm

    zero = jnp.zeros((S, NT), jnp.float32)
    O = (jnp.where(k1, w1, zero) + jnp.where(k2, w2, zero)
         + jnp.where(k3, w3, zero))

    p2 = p2_ref[0]
    interp = jnp.dot(p2, O, preferred_element_type=jnp.float32)

    p1 = p1_ref[0]
    w1a = w1_ref[:, :D1]
    w1b = w1_ref[:, D1:]
    h = (jnp.dot(w1a, p1, preferred_element_type=jnp.float32)
         + jnp.dot(w1b, interp, preferred_element_type=jnp.float32)
         + b1_ref[:, :1])
    h = jnp.maximum(h, 0.0)
    out = jnp.dot(w2_ref[...], h, preferred_element_type=jnp.float32) + b2_ref[:, :1]
    out_ref[0] = jnp.maximum(out, 0.0)


def kernel(xyz1, xyz2, points1, points2, W1, b1, W2, b2):
    x1p = jnp.pad(xyz1, ((0, 0), (0, 5), (0, 0)))
    x2t = jnp.transpose(xyz2, (0, 2, 1))
    x2t = jnp.pad(x2t, ((0, 0), (0, 0), (0, 5)))
    b1c = jnp.reshape(b1, (H1, 1))
    b2c = jnp.reshape(b2, (H2, 1))

    grid = (B, N // NT)
    out = pl.pallas_call(
        _fp_kernel,
        grid=grid,
        in_specs=[
            pl.BlockSpec((1, 8, NT), lambda b, n: (b, 0, n)),
            pl.BlockSpec((1, S, 8), lambda b, n: (b, 0, 0)),
            pl.BlockSpec((1, D1, NT), lambda b, n: (b, 0, n)),
            pl.BlockSpec((1, D2, S), lambda b, n: (b, 0, 0)),
            pl.BlockSpec((H1, D1 + D2), lambda b, n: (0, 0)),
            pl.BlockSpec((H1, 1), lambda b, n: (0, 0)),
            pl.BlockSpec((H2, H1), lambda b, n: (0, 0)),
            pl.BlockSpec((H2, 1), lambda b, n: (0, 0)),
        ],
        out_specs=pl.BlockSpec((1, H2, NT), lambda b, n: (b, 0, n)),
        out_shape=jax.ShapeDtypeStruct((B, H2, N), jnp.float32),
    )(x1p, x2t, points1, points2, W1, b1c, W2, b2c)
    return out


# speedup vs baseline: 3.6154x; 1.1106x over previous
import jax
import jax.numpy as jnp
from jax.experimental import pallas as pl

B, N, S = 8, 4096, 1024
D1, D2 = 128, 256
H1, H2 = 256, 128
NT = 4096


def _fp_kernel(x1_ref, x2_ref, p1_ref, p2_ref, w1_ref, b1_ref, w2_ref, b2_ref,
               out_ref):
    x1 = x1_ref[0]
    x2 = x2_ref[0]
    sqn1 = jnp.sum(x1 * x1, axis=0, keepdims=True)
    sqn2 = jnp.sum(x2 * x2, axis=1, keepdims=True)
    dot = jnp.dot(x2, x1, preferred_element_type=jnp.float32)
    dist = sqn2 + sqn1 - 2.0 * dot

    big = jnp.float32(jnp.inf)

    def take_min(d):
        m = jnp.min(d, axis=0, keepdims=True)
        msk = d == m
        d = jnp.where(msk, big, d)
        return m, msk, d

    m1, k1, dist = take_min(dist)
    m2, k2, dist = take_min(dist)
    m3, k3, dist = take_min(dist)

    r1 = 1.0 / (m1 + 1e-8)
    r2 = 1.0 / (m2 + 1e-8)
    r3 = 1.0 / (m3 + 1e-8)
    norm = r1 + r2 + r3
    w1 = r1 / norm
    w2 = r2 / norm
    w3 = r3 / norm

    zero = jnp.zeros((S, NT), jnp.float32)
    O = jnp.where(k1, w1, jnp.where(k2, w2, jnp.where(k3, w3, zero)))

    p2 = p2_ref[0]
    interp = jnp.dot(p2, O, preferred_element_type=jnp.float32)

    p1 = p1_ref[0]
    w1a = w1_ref[:, :D1]
    w1b = w1_ref[:, D1:]
    h = (jnp.dot(w1a, p1, preferred_element_type=jnp.float32)
         + jnp.dot(w1b, interp, preferred_element_type=jnp.float32)
         + b1_ref[:, :1])
    h = jnp.maximum(h, 0.0)
    out = jnp.dot(w2_ref[...], h, preferred_element_type=jnp.float32) + b2_ref[:, :1]
    out_ref[0] = jnp.maximum(out, 0.0)


def kernel(xyz1, xyz2, points1, points2, W1, b1, W2, b2):
    x1p = jnp.pad(xyz1, ((0, 0), (0, 5), (0, 0)))
    x2t = jnp.transpose(xyz2, (0, 2, 1))
    x2t = jnp.pad(x2t, ((0, 0), (0, 0), (0, 5)))
    b1c = jnp.reshape(b1, (H1, 1))
    b2c = jnp.reshape(b2, (H2, 1))

    grid = (B, N // NT)
    out = pl.pallas_call(
        _fp_kernel,
        grid=grid,
        in_specs=[
            pl.BlockSpec((1, 8, NT), lambda b, n: (b, 0, n)),
            pl.BlockSpec((1, S, 8), lambda b, n: (b, 0, 0)),
            pl.BlockSpec((1, D1, NT), lambda b, n: (b, 0, n)),
            pl.BlockSpec((1, D2, S), lambda b, n: (b, 0, 0)),
            pl.BlockSpec((H1, D1 + D2), lambda b, n: (0, 0)),
            pl.BlockSpec((H1, 1), lambda b, n: (0, 0)),
            pl.BlockSpec((H2, H1), lambda b, n: (0, 0)),
            pl.BlockSpec((H2, 1), lambda b, n: (0, 0)),
        ],
        out_specs=pl.BlockSpec((1, H2, NT), lambda b, n: (b, 0, n)),
        out_shape=jax.ShapeDtypeStruct((B, H2, N), jnp.float32),
    )(x1p, x2t, points1, points2, W1, b1c, W2, b2c)
    return out


# speedup vs baseline: 3.6769x; 1.0170x over previous
import jax
import jax.numpy as jnp
from jax.experimental import pallas as pl

B, N, S = 8, 4096, 1024
D1, D2 = 128, 256
H1, H2 = 256, 128
NT = 4096


def _fp_kernel(x1_ref, x2_ref, p1_ref, p2_ref, w1_ref, b1_ref, w2_ref, b2_ref,
               out_ref):
    x1 = x1_ref[0]
    x2 = x2_ref[0]
    sqn1 = jnp.sum(x1 * x1, axis=0, keepdims=True)
    sqn2 = 0.25 * jnp.sum(x2 * x2, axis=1, keepdims=True)
    dot = jnp.dot(x2, x1, preferred_element_type=jnp.float32)
    dist = sqn2 + dot

    big = jnp.float32(jnp.inf)

    def take_min(d):
        m = jnp.min(d, axis=0, keepdims=True)
        msk = d == m
        d = jnp.where(msk, big, d)
        return m, msk, d

    m1, k1, dist = take_min(dist)
    m2, k2, dist = take_min(dist)
    m3, k3, dist = take_min(dist)

    r1 = 1.0 / (m1 + sqn1 + 1e-8)
    r2 = 1.0 / (m2 + sqn1 + 1e-8)
    r3 = 1.0 / (m3 + sqn1 + 1e-8)
    norm = r1 + r2 + r3
    w1 = r1 / norm
    w2 = r2 / norm
    w3 = r3 / norm

    zero = jnp.zeros((S, NT), jnp.float32)
    O = jnp.where(k1, w1, jnp.where(k2, w2, jnp.where(k3, w3, zero)))

    p2 = p2_ref[0]
    interp = jnp.dot(p2, O, preferred_element_type=jnp.float32)

    p1 = p1_ref[0]
    w1a = w1_ref[:, :D1]
    w1b = w1_ref[:, D1:]
    h = (jnp.dot(w1a, p1, preferred_element_type=jnp.float32)
         + jnp.dot(w1b, interp, preferred_element_type=jnp.float32)
         + b1_ref[:, :1])
    h = jnp.maximum(h, 0.0)
    out = jnp.dot(w2_ref[...], h, preferred_element_type=jnp.float32) + b2_ref[:, :1]
    out_ref[0] = jnp.maximum(out, 0.0)


def kernel(xyz1, xyz2, points1, points2, W1, b1, W2, b2):
    x1p = jnp.pad(xyz1, ((0, 0), (0, 5), (0, 0)))
    x2t = jnp.transpose(-2.0 * xyz2, (0, 2, 1))
    x2t = jnp.pad(x2t, ((0, 0), (0, 0), (0, 5)))
    b1c = jnp.reshape(b1, (H1, 1))
    b2c = jnp.reshape(b2, (H2, 1))

    grid = (B, N // NT)
    out = pl.pallas_call(
        _fp_kernel,
        grid=grid,
        in_specs=[
            pl.BlockSpec((1, 8, NT), lambda b, n: (b, 0, n)),
            pl.BlockSpec((1, S, 8), lambda b, n: (b, 0, 0)),
            pl.BlockSpec((1, D1, NT), lambda b, n: (b, 0, n)),
            pl.BlockSpec((1, D2, S), lambda b, n: (b, 0, 0)),
            pl.BlockSpec((H1, D1 + D2), lambda b, n: (0, 0)),
            pl.BlockSpec((H1, 1), lambda b, n: (0, 0)),
            pl.BlockSpec((H2, H1), lambda b, n: (0, 0)),
            pl.BlockSpec((H2, 1), lambda b, n: (0, 0)),
        ],
        out_specs=pl.BlockSpec((1, H2, NT), lambda b, n: (b, 0, n)),
        out_shape=jax.ShapeDtypeStruct((B, H2, N), jnp.float32),
    )(x1p, x2t, points1, points2, W1, b1c, W2, b2c)
    return out
